# baseline (speedup 1.0000x reference)
import functools

import jax
import jax.numpy as jnp
from jax import lax
from jax.experimental import pallas as pl
from jax.experimental.pallas import tpu as pltpu
from jax.experimental.pallas import tpu_sc as plsc

VOCAB = 1000000
DIM = 32
BATCH = 16384
FIELDS = 26

TOTAL = BATCH * FIELDS
CHUNK = 128


def _sc_gather(table, idx2d, *, num_workers, chunks_per_worker):
    mesh = plsc.VectorSubcoreMesh(core_axis_name="c", subcore_axis_name="s")
    rows_per_worker = chunks_per_worker * CHUNK

    @functools.partial(
        pl.kernel,
        mesh=mesh,
        compiler_params=pltpu.CompilerParams(use_tc_tiling_on_sc=False),
        out_type=jax.ShapeDtypeStruct((TOTAL, DIM), jnp.float32),
        scratch_types=[
            pltpu.VMEM((chunks_per_worker, CHUNK), jnp.int32),
            pltpu.VMEM((CHUNK, DIM), jnp.float32),
            pltpu.VMEM((CHUNK, DIM), jnp.float32),
            pltpu.SemaphoreType.DMA,
            pltpu.SemaphoreType.DMA,
        ],
    )
    def k(table_hbm, idx_hbm, out_hbm, idx_v, buf0, buf1, sem0, sem1):
        nc = 2
        wid = lax.axis_index("s") * nc + lax.axis_index("c")
        chunk_base = wid * chunks_per_worker
        row_base = wid * rows_per_worker
        pltpu.sync_copy(idx_hbm.at[pl.ds(chunk_base, chunks_per_worker)], idx_v)

        bufs = (buf0, buf1)
        sems = (sem0, sem1)

        cp0 = pltpu.async_copy(table_hbm.at[idx_v.at[0]], buf0, sem0)

        def body(j, carry):
            del carry
            @pl.when(j + 1 < chunks_per_worker)
            def _():
                for par in range(2):
                    @pl.when((j + 1) % 2 == par)
                    def _():
                        pltpu.async_copy(
                            table_hbm.at[idx_v.at[j + 1]], bufs[par], sems[par]
                        )

            for par in range(2):
                @pl.when(j % 2 == par)
                def _():
                    pltpu.make_async_copy(
                        table_hbm.at[idx_v.at[j]], bufs[par], sems[par]
                    ).wait()
                    pltpu.sync_copy(
                        bufs[par], out_hbm.at[pl.ds(row_base + j * CHUNK, CHUNK)]
                    )
            return 0

        del cp0
        lax.fori_loop(0, chunks_per_worker, body, 0, unroll=2)

    return k(table, idx2d)


def _tc_transform(emb_fm, w, b2d):
    bblk = 2048
    nb = BATCH // bblk

    def body(e_ref, w_ref, b_ref, o_ref):
        z = jax.lax.dot_general(
            w_ref[...], e_ref[...], (((1,), (1,)), ((), ())),
            preferred_element_type=jnp.float32,
        )
        o_ref[...] = jnp.maximum(z + b_ref[...][:, 0:1], 0.0)[None]

    return pl.pallas_call(
        body,
        grid=(FIELDS, nb),
        in_specs=[
            pl.BlockSpec((bblk, DIM), lambda f, ib: (f * nb + ib, 0)),
            pl.BlockSpec((DIM, DIM), lambda f, ib: (0, 0)),
            pl.BlockSpec((DIM, 128), lambda f, ib: (0, 0)),
        ],
        out_specs=pl.BlockSpec((1, DIM, bblk), lambda f, ib: (f, 0, ib)),
        out_shape=jax.ShapeDtypeStruct((FIELDS, DIM, BATCH), jnp.float32),
    )(emb_fm, w, b2d)


def kernel(x, table, W, b):
    num_workers = 32
    chunks_per_worker = TOTAL // (num_workers * CHUNK)
    idx2d = x.T.reshape(num_workers * chunks_per_worker, CHUNK).astype(jnp.int32)
    emb_fm = _sc_gather(table, idx2d, num_workers=num_workers,
                        chunks_per_worker=chunks_per_worker)
    b2d = jnp.tile(b.reshape(DIM, 1), (1, 128))
    out3 = _tc_transform(emb_fm, W, b2d)
    return out3.transpose(2, 0, 1)


# speedup vs baseline: 1.8375x; 1.8375x over previous
import functools

import jax
import jax.numpy as jnp
from jax import lax
from jax.experimental import pallas as pl
from jax.experimental.pallas import tpu as pltpu
from jax.experimental.pallas import tpu_sc as plsc

VOCAB = 1000000
DIM = 32
BATCH = 16384
FIELDS = 26

TOTAL = BATCH * FIELDS
CHUNK = 128
NCHUNKS = TOTAL // CHUNK
PACK = 128 // DIM
VPACK = VOCAB // PACK


def _tc_pack_table(table_t):
    sub = 256
    nsub = 32
    vb = sub * nsub
    grid = (VOCAB + vb - 1) // vb

    def body(t_ref, o_ref):
        k_io = lax.broadcasted_iota(jnp.int32, (sub // PACK, sub), 0)
        r_io = lax.broadcasted_iota(jnp.int32, (sub // PACK, sub), 1)
        qs = [
            jnp.where(r_io == PACK * k_io + c, 1.0, 0.0).astype(jnp.float32)
            for c in range(PACK)
        ]
        for s in range(nsub):
            tts = t_ref[:, s * sub:(s + 1) * sub]
            for c in range(PACK):
                o = lax.dot_general(qs[c], tts, (((1,), (1,)), ((), ())),
                                    preferred_element_type=jnp.float32)
                o_ref[s * (sub // PACK):(s + 1) * (sub // PACK),
                      DIM * c:DIM * (c + 1)] = o

    return pl.pallas_call(
        body,
        grid=(grid,),
        in_specs=[pl.BlockSpec((DIM, vb), lambda i: (0, i))],
        out_specs=pl.BlockSpec((vb // PACK, 128), lambda i: (i, 0)),
        out_shape=jax.ShapeDtypeStruct((VPACK, 128), jnp.float32),
    )(table_t)


def _sc_gather(table, idx2d, *, chunks_per_worker):
    mesh = plsc.VectorSubcoreMesh(core_axis_name="c", subcore_axis_name="s")

    @functools.partial(
        pl.kernel,
        mesh=mesh,
        compiler_params=pltpu.CompilerParams(use_tc_tiling_on_sc=False),
        out_type=jax.ShapeDtypeStruct((NCHUNKS, CHUNK, DIM), jnp.float32),
        scratch_types=[
            pltpu.VMEM((chunks_per_worker, CHUNK), jnp.int32),
            pltpu.VMEM((CHUNK, DIM), jnp.float32),
            pltpu.VMEM((CHUNK, DIM), jnp.float32),
            pltpu.SemaphoreType.DMA,
            pltpu.SemaphoreType.DMA,
            pltpu.SemaphoreType.DMA,
            pltpu.SemaphoreType.DMA,
        ],
    )
    def k(table_hbm, idx_hbm, out_hbm, idx_v, buf0, buf1,
          gsem0, gsem1, wsem0, wsem1):
        nc = 2
        wid = lax.axis_index("s") * nc + lax.axis_index("c")
        chunk_base = wid * chunks_per_worker
        pltpu.sync_copy(idx_hbm.at[pl.ds(chunk_base, chunks_per_worker)], idx_v)

        bufs = (buf0, buf1)
        gsems = (gsem0, gsem1)
        wsems = (wsem0, wsem1)

        pltpu.async_copy(table_hbm.at[idx_v.at[0]], buf0, gsem0)

        def body(j, carry):
            del carry

            @pl.when(j + 1 < chunks_per_worker)
            def _():
                for par in range(2):
                    @pl.when((j + 1) % 2 == par)
                    def _():
                        pltpu.async_copy(
                            table_hbm.at[idx_v.at[j + 1]], bufs[par], gsems[par]
                        )

            for par in range(2):
                @pl.when(j % 2 == par)
                def _():
                    pltpu.make_async_copy(
                        table_hbm.at[idx_v.at[j]], bufs[par], gsems[par]
                    ).wait()

                    @pl.when(j >= 2)
                    def _():
                        pltpu.make_async_copy(
                            bufs[par], out_hbm.at[0], wsems[par]
                        ).wait()

                    pltpu.async_copy(
                        bufs[par], out_hbm.at[chunk_base + j], wsems[par]
                    )
            return 0

        lax.fori_loop(0, chunks_per_worker, body, 0)

        for par in range(2):
            pltpu.make_async_copy(bufs[par], out_hbm.at[0], wsems[par]).wait()

    return k(table, idx2d)


def _tc_out(emb_pk, w, b2d):
    bblk = 2048
    nb = BATCH // bblk
    rblk = bblk // PACK

    def body(e_ref, w_ref, b_ref, o_ref):
        e = e_ref[...]
        wv = w_ref[...]
        bv = b_ref[...]
        for c in range(PACK):
            e_c = e[:, DIM * c:DIM * (c + 1)]
            z = lax.dot_general(wv, e_c, (((1,), (1,)), ((), ())),
                                preferred_element_type=jnp.float32)
            o_ref[0, :, c * rblk:(c + 1) * rblk] = jnp.maximum(z + bv, 0.0)

    return pl.pallas_call(
        body,
        grid=(FIELDS, nb),
        in_specs=[
            pl.BlockSpec((rblk, 128), lambda f, ib: (f * nb + ib, 0)),
            pl.BlockSpec((DIM, DIM), lambda f, ib: (0, 0)),
            pl.BlockSpec((DIM, rblk), lambda f, ib: (0, 0)),
        ],
        out_specs=pl.BlockSpec((1, DIM, bblk), lambda f, ib: (f, 0, ib)),
        out_shape=jax.ShapeDtypeStruct((FIELDS, DIM, BATCH), jnp.float32),
    )(emb_pk, w, b2d)


def kernel(x, table, W, b):
    num_workers = 32
    chunks_per_worker = NCHUNKS // num_workers
    xt4 = x.T.reshape(FIELDS, BATCH // 2048, PACK, 512).astype(jnp.int32)
    idx2d = xt4.transpose(0, 1, 3, 2).reshape(NCHUNKS, CHUNK)
    t128 = _tc_pack_table(table.T)
    table_lin = t128.reshape(VOCAB, DIM)
    emb3 = _sc_gather(table_lin, idx2d, chunks_per_worker=chunks_per_worker)
    emb_pk = emb3.reshape(TOTAL // PACK, 128)
    b2d = jnp.tile(b.reshape(DIM, 1), (1, 512))
    out3 = _tc_out(emb_pk, W, b2d)
    return out3.transpose(2, 0, 1)


# speedup vs baseline: 2.2321x; 1.2148x over previous
import functools

import jax
import jax.numpy as jnp
from jax import lax
from jax.experimental import pallas as pl
from jax.experimental.pallas import tpu as pltpu
from jax.experimental.pallas import tpu_sc as plsc

VOCAB = 1000000
DIM = 32
BATCH = 16384
FIELDS = 26

TOTAL = BATCH * FIELDS
CHUNK = 128
NCHUNKS = TOTAL // CHUNK
PACK = 128 // DIM
VPACK = VOCAB // PACK


def _tc_pack_table(table_t):
    sub = 256
    nsub = 32
    vb = sub * nsub
    grid = (VOCAB + vb - 1) // vb

    def body(t_ref, o_ref):
        k_io = lax.broadcasted_iota(jnp.int32, (sub // PACK, sub), 0)
        r_io = lax.broadcasted_iota(jnp.int32, (sub // PACK, sub), 1)
        qbig = jnp.where(r_io // PACK == k_io, 1.0, 0.0).astype(jnp.float32)
        lane = lax.broadcasted_iota(jnp.int32, (DIM, sub), 1)
        masks = [(lane % PACK == c).astype(jnp.float32) for c in range(PACK)]
        for s in range(nsub):
            tts = t_ref[:, s * sub:(s + 1) * sub]
            tt4 = jnp.concatenate([tts * m for m in masks], axis=0)
            o = lax.dot_general(qbig, tt4, (((1,), (1,)), ((), ())),
                                preferred_element_type=jnp.float32)
            o_ref[s * (sub // PACK):(s + 1) * (sub // PACK), :] = o

    return pl.pallas_call(
        body,
        grid=(grid,),
        in_specs=[pl.BlockSpec((DIM, vb), lambda i: (0, i))],
        out_specs=pl.BlockSpec((vb // PACK, 128), lambda i: (i, 0)),
        out_shape=jax.ShapeDtypeStruct((VPACK, 128), jnp.float32),
    )(table_t)


def _sc_gather(table, idx2d, *, chunks_per_worker):
    mesh = plsc.VectorSubcoreMesh(core_axis_name="c", subcore_axis_name="s")

    @functools.partial(
        pl.kernel,
        mesh=mesh,
        compiler_params=pltpu.CompilerParams(use_tc_tiling_on_sc=False),
        out_type=jax.ShapeDtypeStruct((NCHUNKS, CHUNK, DIM), jnp.float32),
        scratch_types=[
            pltpu.VMEM((chunks_per_worker, CHUNK), jnp.int32),
            pltpu.VMEM((CHUNK, DIM), jnp.float32),
            pltpu.VMEM((CHUNK, DIM), jnp.float32),
            pltpu.SemaphoreType.DMA,
            pltpu.SemaphoreType.DMA,
            pltpu.SemaphoreType.DMA,
            pltpu.SemaphoreType.DMA,
        ],
    )
    def k(table_hbm, idx_hbm, out_hbm, idx_v, buf0, buf1,
          gsem0, gsem1, wsem0, wsem1):
        nc = 2
        wid = lax.axis_index("s") * nc + lax.axis_index("c")
        chunk_base = wid * chunks_per_worker
        pltpu.sync_copy(idx_hbm.at[pl.ds(chunk_base, chunks_per_worker)], idx_v)

        bufs = (buf0, buf1)
        gsems = (gsem0, gsem1)
        wsems = (wsem0, wsem1)

        pltpu.async_copy(table_hbm.at[idx_v.at[0]], buf0, gsem0)

        def body(j, carry):
            del carry

            @pl.when(j + 1 < chunks_per_worker)
            def _():
                for par in range(2):
                    @pl.when((j + 1) % 2 == par)
                    def _():
                        pltpu.async_copy(
                            table_hbm.at[idx_v.at[j + 1]], bufs[par], gsems[par]
                        )

            for par in range(2):
                @pl.when(j % 2 == par)
                def _():
                    pltpu.make_async_copy(
                        table_hbm.at[idx_v.at[j]], bufs[par], gsems[par]
                    ).wait()

                    @pl.when(j >= 2)
                    def _():
                        pltpu.make_async_copy(
                            bufs[par], out_hbm.at[0], wsems[par]
                        ).wait()

                    pltpu.async_copy(
                        bufs[par], out_hbm.at[chunk_base + j], wsems[par]
                    )
            return 0

        lax.fori_loop(0, chunks_per_worker, body, 0)

        for par in range(2):
            pltpu.make_async_copy(bufs[par], out_hbm.at[0], wsems[par]).wait()

    return k(table, idx2d)


def _tc_out(emb_pk, w, b2d):
    bblk = 4096
    nb = BATCH // bblk
    rblk = bblk // PACK

    def body(e_ref, w_ref, b_ref, o_ref):
        e = e_ref[...]
        wv = w_ref[...]
        bv = b_ref[...]
        for c in range(PACK):
            e_c = e[:, DIM * c:DIM * (c + 1)]
            z = lax.dot_general(wv, e_c, (((1,), (1,)), ((), ())),
                                preferred_element_type=jnp.float32)
            o_ref[0, :, c * rblk:(c + 1) * rblk] = jnp.maximum(z + bv, 0.0)

    return pl.pallas_call(
        body,
        grid=(FIELDS, nb),
        in_specs=[
            pl.BlockSpec((rblk, 128), lambda f, ib: (f * nb + ib, 0)),
            pl.BlockSpec((DIM, DIM), lambda f, ib: (0, 0)),
            pl.BlockSpec((DIM, rblk), lambda f, ib: (0, 0)),
        ],
        out_specs=pl.BlockSpec((1, DIM, bblk), lambda f, ib: (f, 0, ib)),
        out_shape=jax.ShapeDtypeStruct((FIELDS, DIM, BATCH), jnp.float32),
    )(emb_pk, w, b2d)


def kernel(x, table, W, b):
    num_workers = 32
    chunks_per_worker = NCHUNKS // num_workers
    xt4 = x.T.reshape(FIELDS, BATCH // 4096, PACK, 1024).astype(jnp.int32)
    idx2d = xt4.transpose(0, 1, 3, 2).reshape(NCHUNKS, CHUNK)
    t128 = _tc_pack_table(table.T)
    table_lin = t128.reshape(VOCAB, DIM)
    emb3 = _sc_gather(table_lin, idx2d, chunks_per_worker=chunks_per_worker)
    emb_pk = emb3.reshape(TOTAL // PACK, 128)
    b2d = jnp.tile(b.reshape(DIM, 1), (1, 1024))
    out3 = _tc_out(emb_pk, W, b2d)
    return out3.transpose(2, 0, 1)


# speedup vs baseline: 2.2360x; 1.0017x over previous
import functools

import jax
import jax.numpy as jnp
from jax import lax
from jax.experimental import pallas as pl
from jax.experimental.pallas import tpu as pltpu
from jax.experimental.pallas import tpu_sc as plsc

VOCAB = 1000000
DIM = 32
BATCH = 16384
FIELDS = 26

TOTAL = BATCH * FIELDS
CHUNK = 128
NCHUNKS = TOTAL // CHUNK
PACK = 128 // DIM
VPACK = VOCAB // PACK


def _tc_pack_table(table_t):
    sub = 256
    nsub = 32
    vb = sub * nsub
    grid = (VOCAB + vb - 1) // vb

    def body(t_ref, o_ref):
        k_io = lax.broadcasted_iota(jnp.int32, (sub // PACK, sub), 0)
        r_io = lax.broadcasted_iota(jnp.int32, (sub // PACK, sub), 1)
        qbig = jnp.where(r_io // PACK == k_io, 1.0, 0.0).astype(jnp.float32)
        lane = lax.broadcasted_iota(jnp.int32, (DIM, sub), 1)
        masks = [(lane % PACK == c).astype(jnp.float32) for c in range(PACK)]
        for s in range(nsub):
            tts = t_ref[:, s * sub:(s + 1) * sub]
            tt4 = jnp.concatenate([tts * m for m in masks], axis=0)
            o = lax.dot_general(qbig, tt4, (((1,), (1,)), ((), ())),
                                preferred_element_type=jnp.float32)
            o_ref[s * (sub // PACK):(s + 1) * (sub // PACK), :] = o

    return pl.pallas_call(
        body,
        grid=(grid,),
        in_specs=[pl.BlockSpec((DIM, vb), lambda i: (0, i))],
        out_specs=pl.BlockSpec((vb // PACK, 128), lambda i: (i, 0)),
        out_shape=jax.ShapeDtypeStruct((VPACK, 128), jnp.float32),
    )(table_t)


def _sc_gather(table, idx2d, *, chunks_per_worker):
    mesh = plsc.VectorSubcoreMesh(core_axis_name="c", subcore_axis_name="s")

    @functools.partial(
        pl.kernel,
        mesh=mesh,
        compiler_params=pltpu.CompilerParams(use_tc_tiling_on_sc=False),
        out_type=jax.ShapeDtypeStruct((NCHUNKS, CHUNK, DIM), jnp.float32),
        scratch_types=[
            pltpu.VMEM((chunks_per_worker, CHUNK), jnp.int32),
            pltpu.VMEM((CHUNK, DIM), jnp.float32),
            pltpu.VMEM((CHUNK, DIM), jnp.float32),
            pltpu.SemaphoreType.DMA,
            pltpu.SemaphoreType.DMA,
            pltpu.SemaphoreType.DMA,
            pltpu.SemaphoreType.DMA,
        ],
    )
    def k(table_hbm, idx_hbm, out_hbm, idx_v, buf0, buf1,
          gsem0, gsem1, wsem0, wsem1):
        nc = 2
        wid = lax.axis_index("s") * nc + lax.axis_index("c")
        chunk_base = wid * chunks_per_worker
        pltpu.sync_copy(idx_hbm.at[pl.ds(chunk_base, chunks_per_worker)], idx_v)

        bufs = (buf0, buf1)
        gsems = (gsem0, gsem1)
        wsems = (wsem0, wsem1)

        pltpu.async_copy(table_hbm.at[idx_v.at[0]], buf0, gsem0)

        def body(j, carry):
            del carry

            @pl.when(j + 1 < chunks_per_worker)
            def _():
                for par in range(2):
                    @pl.when((j + 1) % 2 == par)
                    def _():
                        pltpu.async_copy(
                            table_hbm.at[idx_v.at[j + 1]], bufs[par], gsems[par]
                        )

            for par in range(2):
                @pl.when(j % 2 == par)
                def _():
                    pltpu.make_async_copy(
                        table_hbm.at[idx_v.at[j]], bufs[par], gsems[par]
                    ).wait()

                    @pl.when(j >= 2)
                    def _():
                        pltpu.make_async_copy(
                            bufs[par], out_hbm.at[0], wsems[par]
                        ).wait()

                    pltpu.async_copy(
                        bufs[par], out_hbm.at[chunk_base + j], wsems[par]
                    )
            return 0

        lax.fori_loop(0, chunks_per_worker, body, 0)

        for par in range(2):
            pltpu.make_async_copy(bufs[par], out_hbm.at[0], wsems[par]).wait()

    return k(table, idx2d)


def _tc_out(emb_pk, w, b2d):
    bblk = 4096
    nb = BATCH // bblk
    rblk = bblk // PACK

    def body(e_ref, w_ref, b_ref, o_ref):
        e = e_ref[...]
        e_cat = jnp.concatenate(
            [e[:, DIM * c:DIM * (c + 1)] for c in range(PACK)], axis=0)
        z = lax.dot_general(w_ref[...], e_cat, (((1,), (1,)), ((), ())),
                            preferred_element_type=jnp.float32)
        o_ref[...] = jnp.maximum(z + b_ref[...], 0.0)[None]

    return pl.pallas_call(
        body,
        grid=(FIELDS, nb),
        in_specs=[
            pl.BlockSpec((rblk, 128), lambda f, ib: (f * nb + ib, 0)),
            pl.BlockSpec((DIM, DIM), lambda f, ib: (0, 0)),
            pl.BlockSpec((DIM, bblk), lambda f, ib: (0, 0)),
        ],
        out_specs=pl.BlockSpec((1, DIM, bblk), lambda f, ib: (f, 0, ib)),
        out_shape=jax.ShapeDtypeStruct((FIELDS, DIM, BATCH), jnp.float32),
    )(emb_pk, w, b2d)


def kernel(x, table, W, b):
    num_workers = 32
    chunks_per_worker = NCHUNKS // num_workers
    xt4 = x.T.reshape(FIELDS, BATCH // 4096, PACK, 1024).astype(jnp.int32)
    idx2d = xt4.transpose(0, 1, 3, 2).reshape(NCHUNKS, CHUNK)
    t128 = _tc_pack_table(table.T)
    table_lin = t128.reshape(VOCAB, DIM)
    emb3 = _sc_gather(table_lin, idx2d, chunks_per_worker=chunks_per_worker)
    emb_pk = emb3.reshape(TOTAL // PACK, 128)
    b2d = jnp.tile(b.reshape(DIM, 1), (1, 4096))
    out3 = _tc_out(emb_pk, W, b2d)
    return out3.transpose(2, 0, 1)


# speedup vs baseline: 2.2860x; 1.0224x over previous
import functools

import jax
import jax.numpy as jnp
from jax import lax
from jax.experimental import pallas as pl
from jax.experimental.pallas import tpu as pltpu
from jax.experimental.pallas import tpu_sc as plsc

VOCAB = 1000000
DIM = 32
BATCH = 16384
FIELDS = 26

TOTAL = BATCH * FIELDS
CHUNK = 128
NCHUNKS = TOTAL // CHUNK
PACK = 128 // DIM
VPACK = VOCAB // PACK


def _tc_pack_table(table_t):
    sub = 256
    nsub = 32
    vb = sub * nsub
    grid = (VOCAB + vb - 1) // vb

    def body(t_ref, o_ref):
        k_io = lax.broadcasted_iota(jnp.int32, (sub // PACK, sub), 0)
        r_io = lax.broadcasted_iota(jnp.int32, (sub // PACK, sub), 1)
        qbig = jnp.where(r_io // PACK == k_io, 1.0, 0.0).astype(jnp.float32)
        lane = lax.broadcasted_iota(jnp.int32, (DIM, sub), 1)
        masks = [(lane % PACK == c).astype(jnp.float32) for c in range(PACK)]
        for s in range(nsub):
            tts = t_ref[:, s * sub:(s + 1) * sub]
            tt4 = jnp.concatenate([tts * m for m in masks], axis=0)
            o = lax.dot_general(qbig, tt4, (((1,), (1,)), ((), ())),
                                preferred_element_type=jnp.float32)
            o_ref[s * (sub // PACK):(s + 1) * (sub // PACK), :] = o

    return pl.pallas_call(
        body,
        grid=(grid,),
        in_specs=[pl.BlockSpec((DIM, vb), lambda i: (0, i))],
        out_specs=pl.BlockSpec((vb // PACK, 128), lambda i: (i, 0)),
        out_shape=jax.ShapeDtypeStruct((VPACK, 128), jnp.float32),
    )(table_t)


def _sc_gather(table, idx2d, *, chunks_per_worker):
    mesh = plsc.VectorSubcoreMesh(core_axis_name="c", subcore_axis_name="s")

    @functools.partial(
        pl.kernel,
        mesh=mesh,
        compiler_params=pltpu.CompilerParams(use_tc_tiling_on_sc=False),
        out_type=jax.ShapeDtypeStruct((NCHUNKS, CHUNK, DIM), jnp.float32),
        scratch_types=[
            pltpu.VMEM((chunks_per_worker, CHUNK), jnp.int32),
            pltpu.VMEM((CHUNK, DIM), jnp.float32),
            pltpu.VMEM((CHUNK, DIM), jnp.float32),
            pltpu.SemaphoreType.DMA,
            pltpu.SemaphoreType.DMA,
            pltpu.SemaphoreType.DMA,
            pltpu.SemaphoreType.DMA,
        ],
    )
    def k(table_hbm, idx_hbm, out_hbm, idx_v, buf0, buf1,
          gsem0, gsem1, wsem0, wsem1):
        nc = 2
        wid = lax.axis_index("s") * nc + lax.axis_index("c")
        chunk_base = wid * chunks_per_worker
        pltpu.sync_copy(idx_hbm.at[pl.ds(chunk_base, chunks_per_worker)], idx_v)

        bufs = (buf0, buf1)
        gsems = (gsem0, gsem1)
        wsems = (wsem0, wsem1)

        pltpu.async_copy(table_hbm.at[idx_v.at[0]], buf0, gsem0)

        def body(j, carry):
            del carry

            @pl.when(j + 1 < chunks_per_worker)
            def _():
                for par in range(2):
                    @pl.when((j + 1) % 2 == par)
                    def _():
                        pltpu.async_copy(
                            table_hbm.at[idx_v.at[j + 1]], bufs[par], gsems[par]
                        )

            for par in range(2):
                @pl.when(j % 2 == par)
                def _():
                    pltpu.make_async_copy(
                        table_hbm.at[idx_v.at[j]], bufs[par], gsems[par]
                    ).wait()

                    @pl.when(j >= 2)
                    def _():
                        pltpu.make_async_copy(
                            bufs[par], out_hbm.at[0], wsems[par]
                        ).wait()

                    pltpu.async_copy(
                        bufs[par], out_hbm.at[chunk_base + j], wsems[par]
                    )
            return 0

        lax.fori_loop(0, chunks_per_worker, body, 0)

        for par in range(2):
            pltpu.make_async_copy(bufs[par], out_hbm.at[0], wsems[par]).wait()

    return k(table, idx2d)


def _tc_out(emb_pk, w, b2d):
    bblk = 4096
    nb = BATCH // bblk
    rblk = bblk // PACK

    def body(e_ref, w4_ref, b_ref, o_ref):
        z = lax.dot_general(w4_ref[...], e_ref[...], (((1,), (1,)), ((), ())),
                            preferred_element_type=jnp.float32)
        z = jnp.maximum(z + b_ref[...][:, 0:1], 0.0)
        for c in range(PACK):
            o_ref[0, :, c * rblk:(c + 1) * rblk] = z[DIM * c:DIM * (c + 1), :]

    return pl.pallas_call(
        body,
        grid=(FIELDS, nb),
        in_specs=[
            pl.BlockSpec((rblk, 128), lambda f, ib: (f * nb + ib, 0)),
            pl.BlockSpec((128, 128), lambda f, ib: (0, 0)),
            pl.BlockSpec((128, 128), lambda f, ib: (0, 0)),
        ],
        out_specs=pl.BlockSpec((1, DIM, bblk), lambda f, ib: (f, 0, ib)),
        out_shape=jax.ShapeDtypeStruct((FIELDS, DIM, BATCH), jnp.float32),
    )(emb_pk, w, b2d)


def kernel(x, table, W, b):
    num_workers = 32
    chunks_per_worker = NCHUNKS // num_workers
    xt4 = x.T.reshape(FIELDS, BATCH // 4096, PACK, 1024).astype(jnp.int32)
    idx2d = xt4.transpose(0, 1, 3, 2).reshape(NCHUNKS, CHUNK)
    t128 = _tc_pack_table(table.T)
    table_lin = t128.reshape(VOCAB, DIM)
    emb3 = _sc_gather(table_lin, idx2d, chunks_per_worker=chunks_per_worker)
    emb_pk = emb3.reshape(TOTAL // PACK, 128)
    w4 = jnp.kron(jnp.eye(PACK, dtype=W.dtype), W)
    b4c = jnp.tile(jnp.tile(b, PACK).reshape(128, 1), (1, 128))
    out3 = _tc_out(emb_pk, w4, b4c)
    return out3.transpose(2, 0, 1)
